# baseline (speedup 1.0000x reference)
import functools

import jax
import jax.numpy as jnp
from jax import lax
from jax.experimental import pallas as pl
from jax.experimental.pallas import tpu as pltpu
from jax.experimental.pallas import tpu_sc as plsc


@functools.lru_cache(maxsize=None)
def _gather4(B, D, dtype_name):
    info = plsc.get_sparse_core_info()
    NC, NS = info.num_cores, info.num_subcores
    NW = NC * NS
    assert B % (8 * NW) == 0
    BPW = B // NW
    dtype = jnp.dtype(dtype_name)
    mesh = plsc.VectorSubcoreMesh(core_axis_name="c", subcore_axis_name="s")

    @functools.partial(
        pl.kernel,
        mesh=mesh,
        out_type=[jax.ShapeDtypeStruct((B, D), dtype) for _ in range(4)],
        scratch_types=[
            pltpu.VMEM((BPW,), jnp.int32),
            pltpu.VMEM((BPW, D), dtype),
            pltpu.SemaphoreType.DMA,
        ],
        compiler_params=pltpu.CompilerParams(use_tc_tiling_on_sc=False),
    )
    def gather4(uid, wid, cid, aid, authors, docs,
                u_out, w_out, c_out, a_out, idx_v, rows_v, sem):
        w = lax.axis_index("s") * NC + lax.axis_index("c")
        base = w * BPW
        for idx_hbm, tab, out in ((uid, authors, u_out),
                                  (wid, docs, w_out),
                                  (cid, docs, c_out),
                                  (aid, authors, a_out)):
            pltpu.sync_copy(idx_hbm.at[pl.ds(base, BPW)], idx_v)
            pltpu.async_copy(tab.at[idx_v], rows_v, sem).wait()
            pltpu.sync_copy(rows_v, out.at[pl.ds(base, BPW)])

    return gather4


def _transh_body(hp_ref, rel_ref, w_ref, c_ref, a_ref,
                 wo_ref, co_ref, ao_ref, wr_ref, cr_ref, ar_ref):
    hp = hp_ref[...]
    nrm = jnp.maximum(jnp.sqrt(jnp.sum(hp * hp, axis=-1, keepdims=True)), 1e-12)
    hpn = hp / nrm
    rel = rel_ref[...]
    for k, (e_ref, o_ref, r_ref) in enumerate(
            ((w_ref, wo_ref, wr_ref), (c_ref, co_ref, cr_ref), (a_ref, ao_ref, ar_ref))):
        e = e_ref[...]
        h = hpn[k:k + 1, :]
        proj = jnp.sum(e * h, axis=-1, keepdims=True)
        o_ref[...] = e - proj * h
        r_ref[...] = jnp.broadcast_to(rel[k:k + 1, :], e.shape)


@functools.lru_cache(maxsize=None)
def _transh(B, D, NR, blk):
    big = pl.BlockSpec((blk, D), lambda i: (i, 0))
    small = pl.BlockSpec((NR, D), lambda i: (0, 0))
    return pl.pallas_call(
        _transh_body,
        grid=(B // blk,),
        in_specs=[small, small, big, big, big],
        out_specs=[big] * 6,
        out_shape=[jax.ShapeDtypeStruct((B, D), jnp.float32)] * 6,
    )


def kernel(user_id, wrote, cited, coauthor, author_weight, doc_embs,
           relation_weight, hyper_plane_weight):
    B = user_id.shape[0]
    D = author_weight.shape[1]
    NR = relation_weight.shape[0]
    idx = [x.astype(jnp.int32) for x in (user_id, wrote, cited, coauthor)]
    u, w_raw, c_raw, a_raw = _gather4(B, D, str(author_weight.dtype))(
        *idx, author_weight, doc_embs)
    w_t, c_t, a_t, w_rel, c_rel, a_rel = _transh(B, D, NR, 2048)(
        hyper_plane_weight, relation_weight, w_raw, c_raw, a_raw)
    return (u, w_t, c_t, a_t, w_rel, c_rel, a_rel)
